# Initial kernel scaffold; baseline (speedup 1.0000x reference)
import jax
import jax.numpy as jnp
from jax.experimental import pallas as pl


def kernel(features, edges, edges2, edge_features, W1, b1, W2, b2):
    raise NotImplementedError("write your pallas kernel here")


# speedup vs baseline: 14.0023x; 14.0023x over previous
import functools

import jax
import jax.numpy as jnp
from jax import lax
from jax.experimental import pallas as pl
from jax.experimental.pallas import tpu as pltpu
from jax.experimental.pallas import tpu_sc as plsc

_N = 10000
_E = 320000
_DHID = 128

_NC = 2
_NS = 16
_NW = _NC * _NS
_EPW = _E // _NW
_CH = 80
_NCHUNKS = _EPW // _CH
_NPAD = 10240
_RPT = _NPAD // _NS
_CHD = 10000
_NCHUNKS_D = _EPW // _CHD


def _make_prop():
    mesh = plsc.VectorSubcoreMesh(core_axis_name="c", subcore_axis_name="s")

    @functools.partial(
        pl.kernel, mesh=mesh,
        out_type=jax.ShapeDtypeStruct((_NC * _NPAD, _DHID), jnp.float32),
        scratch_types=[
            pltpu.VMEM((_CH,), jnp.int32),
            pltpu.VMEM((_CH,), jnp.int32),
            pltpu.VMEM((_CH, _DHID), jnp.float32),
            pltpu.VMEM_SHARED((_NPAD, _DHID), jnp.float32),
            pltpu.SemaphoreType.DMA,
        ],
    )
    def k(src, rows, cols, zrows, out, ridx, cidx, rbuf, acc, sem):
        c = lax.axis_index("c")
        s = lax.axis_index("s")
        wid = s * _NC + c

        pltpu.sync_copy(zrows, acc.at[pl.ds(s * _RPT, _RPT)])
        plsc.subcore_barrier()

        def chunk(i, carry):
            base = wid * _EPW + i * _CH
            pltpu.sync_copy(rows.at[pl.ds(base, _CH)], ridx)
            pltpu.async_copy(src.at[ridx], rbuf, sem).wait()
            pltpu.sync_copy(cols.at[pl.ds(base, _CH)], cidx)
            pltpu.sync_copy(rbuf, acc.at[cidx], add=True)
            return carry

        lax.fori_loop(0, _NCHUNKS, chunk, 0)
        plsc.subcore_barrier()

        pltpu.sync_copy(acc.at[pl.ds(s * _RPT, _RPT)],
                        out.at[pl.ds(c * _NPAD + s * _RPT, _RPT)])

    return k


def _make_deg():
    mesh = plsc.VectorSubcoreMesh(core_axis_name="c", subcore_axis_name="s")

    @functools.partial(
        pl.kernel, mesh=mesh,
        out_type=jax.ShapeDtypeStruct((_NW, _NPAD), jnp.float32),
        scratch_types=[
            pltpu.VMEM((_CHD,), jnp.int32),
            pltpu.VMEM((_NPAD,), jnp.float32),
        ],
        compiler_params=pltpu.CompilerParams(needs_layout_passes=False),
    )
    def k(cols, out, cidx, hist):
        c = lax.axis_index("c")
        s = lax.axis_index("s")
        wid = s * _NC + c

        z16 = jnp.zeros((16,), jnp.float32)

        def zero(i, carry):
            hist[pl.ds(i * 16, 16)] = z16
            return carry

        lax.fori_loop(0, _NPAD // 16, zero, 0)

        def chunk(i, carry):
            base = wid * _EPW + i * _CHD
            pltpu.sync_copy(cols.at[pl.ds(base, _CHD)], cidx)

            def vec(j, carry2):
                idx16 = cidx[pl.ds(j * 16, 16)]
                cnt, last = plsc.scan_count(idx16)
                plsc.addupdate_scatter(
                    hist, [idx16], cnt.astype(jnp.float32), mask=last)
                return carry2

            return lax.fori_loop(0, _CHD // 16, vec, carry)

        lax.fori_loop(0, _NCHUNKS_D, chunk, 0)
        pltpu.sync_copy(hist, out.at[wid])

    return k


_prop128 = _make_prop()
_deg_pass = _make_deg()


def _dinv_from(deg_part_ref):
    ones32 = jnp.ones((_NW, 1), jnp.float32)
    deg = lax.dot_general(deg_part_ref[...], ones32,
                          (((0,), (0,)), ((), ())),
                          preferred_element_type=jnp.float32)
    return lax.rsqrt(deg[: _N, :] + 1.0)


def _tc1_body(dp, f, w, g1):
    dinv = _dinv_from(dp)
    g1[...] = jnp.dot(f[...], w[...], preferred_element_type=jnp.float32) * dinv


def _tc2_body(dp, scat1, g1, b1, y):
    dinv = _dinv_from(dp)
    agg = scat1[: _N, :] + scat1[_NPAD : _NPAD + _N, :] + g1[...]
    y[...] = jnp.maximum(agg * dinv + b1[...], 0.0) * dinv


def _tc3_body(dp, scat2, y, w2, b2, out):
    dinv = _dinv_from(dp)
    z = (scat2[: _N, :] + scat2[_NPAD : _NPAD + _N, :] + y[...]) * dinv
    out[...] = jnp.dot(z, w2[...], preferred_element_type=jnp.float32) + b2[...]


def kernel(features, edges, edges2, edge_features, W1, b1, W2, b2):
    del edges2, edge_features
    rows = edges[0]
    cols = edges[1]

    zeros128 = jnp.zeros((_RPT, _DHID), jnp.float32)
    b1_2d = b1.reshape(1, _DHID)
    b2_2d = b2.reshape(1, 3)

    deg_part = _deg_pass(cols)

    g1 = pl.pallas_call(
        _tc1_body,
        out_shape=jax.ShapeDtypeStruct((_N, _DHID), jnp.float32),
    )(deg_part, features, W1)

    scat1 = _prop128(g1, rows, cols, zeros128)

    y = pl.pallas_call(
        _tc2_body,
        out_shape=jax.ShapeDtypeStruct((_N, _DHID), jnp.float32),
    )(deg_part, scat1, g1, b1_2d)

    scat2 = _prop128(y, rows, cols, zeros128)

    out = pl.pallas_call(
        _tc3_body,
        out_shape=jax.ShapeDtypeStruct((_N, 3), jnp.float32),
    )(deg_part, scat2, y, W2, b2_2d)

    return out
